# Initial kernel scaffold; baseline (speedup 1.0000x reference)
import jax
import jax.numpy as jnp
from jax.experimental import pallas as pl


def kernel(xyz_list_0, xyz_list_1, xyz_list_2, x_list_0, x_list_1, x_list_2):
    raise NotImplementedError("write your pallas kernel here")


# speedup vs baseline: 11.8210x; 11.8210x over previous
import functools

import jax
import jax.numpy as jnp
from jax import lax
from jax.experimental import pallas as pl
from jax.experimental.pallas import tpu as pltpu

_K = 3
_BIG = 3.0e38


def _knn_body(q_ref, p_ref, idx_ref, w_ref, *, S):
    q = q_ref[...]
    p = p_ref[...]
    qn = (q[:, 0:1] * q[:, 0:1] + q[:, 1:2] * q[:, 1:2]) + q[:, 2:3] * q[:, 2:3]
    pn = (p[0:1, :] * p[0:1, :] + p[1:2, :] * p[1:2, :]) + p[2:3, :] * p[2:3, :]
    cross = lax.dot_general(q.astype(jnp.bfloat16), p.astype(jnp.bfloat16),
                            (((1,), (0,)), ((), ())),
                            preferred_element_type=jnp.float32)
    d = -2.0 * cross + qn + pn
    iota = lax.broadcasted_iota(jnp.int32, d.shape, 1)
    idxs, vals = [], []
    for _ in range(_K):
        m = jnp.min(d, axis=1, keepdims=True)
        i = jnp.min(jnp.where(d == m, iota, S), axis=1, keepdims=True)
        idxs.append(i)
        vals.append(m)
        d = jnp.where(iota == i, _BIG, d)
    r = [1.0 / (v + 1e-8) for v in vals]
    norm = r[0] + r[1] + r[2]
    base = pl.program_id(0) * S
    idx_ref[...] = jnp.concatenate([i + base for i in idxs], axis=1)
    w_ref[...] = jnp.concatenate([ri / norm for ri in r], axis=1)


def _knn_topk(xyz1, xyz2t, tile_n):
    B, N, _ = xyz1.shape
    S = xyz2t.shape[2]
    grid = (B, N // tile_n)
    return pl.pallas_call(
        functools.partial(_knn_body, S=S),
        grid=grid,
        in_specs=[
            pl.BlockSpec((None, tile_n, 3), lambda b, n: (b, n, 0)),
            pl.BlockSpec((None, 3, S), lambda b, n: (b, 0, 0)),
        ],
        out_specs=[
            pl.BlockSpec((None, tile_n, _K), lambda b, n: (b, n, 0)),
            pl.BlockSpec((None, tile_n, _K), lambda b, n: (b, n, 0)),
        ],
        out_shape=[
            jax.ShapeDtypeStruct((B, N, _K), jnp.int32),
            jax.ShapeDtypeStruct((B, N, _K), jnp.float32),
        ],
    )(xyz1, xyz2t)


def _interp_body(idx_ref, w_ref, p2_ref, out_ref, s1_ref, s2_ref, *, S):
    gidx = idx_ref[...]
    w = w_ref[...]
    base = pl.program_id(0) * S
    iota = lax.broadcasted_iota(jnp.int32, (gidx.shape[0], S), 1) + base
    W = jnp.where(iota == gidx[:, 0:1], w[:, 0:1], 0.0)
    W = W + jnp.where(iota == gidx[:, 1:2], w[:, 1:2], 0.0)
    W = W + jnp.where(iota == gidx[:, 2:3], w[:, 2:3], 0.0)
    p2 = p2_ref[...]
    out = lax.dot_general(p2, W, (((1,), (1,)), ((), ())),
                          preferred_element_type=jnp.float32)
    out_ref[...] = out
    s1_ref[...] = jnp.full((1, 128), jnp.sum(out), jnp.float32)
    s2_ref[...] = jnp.full((1, 128), jnp.sum(out * out), jnp.float32)


def _interp_matmul(idx, w, p2, tile_c, tile_n):
    B, N, _ = idx.shape
    C, S = p2.shape[1], p2.shape[2]
    ct, nt = C // tile_c, N // tile_n
    grid = (B, ct, nt)
    out, s1, s2 = pl.pallas_call(
        functools.partial(_interp_body, S=S),
        grid=grid,
        in_specs=[
            pl.BlockSpec((None, tile_n, _K), lambda b, c, n: (b, n, 0)),
            pl.BlockSpec((None, tile_n, _K), lambda b, c, n: (b, n, 0)),
            pl.BlockSpec((None, tile_c, S), lambda b, c, n: (b, c, 0)),
        ],
        out_specs=[
            pl.BlockSpec((None, tile_c, tile_n), lambda b, c, n: (b, c, n)),
            pl.BlockSpec((None, None, None, 1, 128), lambda b, c, n: (b, c, n, 0, 0)),
            pl.BlockSpec((None, None, None, 1, 128), lambda b, c, n: (b, c, n, 0, 0)),
        ],
        out_shape=[
            jax.ShapeDtypeStruct((B, C, N), jnp.float32),
            jax.ShapeDtypeStruct((B, ct, nt, 1, 128), jnp.float32),
            jax.ShapeDtypeStruct((B, ct, nt, 1, 128), jnp.float32),
        ],
    )(idx, w, p2)
    return out, jnp.sum(s1[..., 0]), jnp.sum(s2[..., 0])


def _mean_scale(s1, s2, n):
    mean = s1 / n
    m2 = s2 - n * mean * mean
    std = jnp.sqrt(m2 / (n - 1))
    scale = 1.0 / (std + 1e-5)
    return jnp.stack([mean, scale]).astype(jnp.float32)


def _norm_body(x1_ref, it_ref, st_ref, out_ref, *, cs):
    c = pl.program_id(2)

    @pl.when(c < cs)
    def _copy():
        out_ref[...] = x1_ref[...]

    @pl.when(c >= cs)
    def _norm():
        out_ref[...] = (it_ref[...] - st_ref[0]) * st_ref[1]


def _norm_concat(x1, interp_t, stats, tile_n):
    B, C1, N = x1.shape
    C2 = interp_t.shape[1]
    tc = 128
    cs = C1 // tc
    ct = (C1 + C2) // tc
    grid = (B, N // tile_n, ct)
    return pl.pallas_call(
        functools.partial(_norm_body, cs=cs),
        grid=grid,
        in_specs=[
            pl.BlockSpec((None, tc, tile_n),
                         lambda b, n, c: (b, jnp.minimum(c, cs - 1), n)),
            pl.BlockSpec((None, tc, tile_n),
                         lambda b, n, c: (b, jnp.maximum(c - cs, 0), n)),
            pl.BlockSpec(memory_space=pltpu.SMEM),
        ],
        out_specs=pl.BlockSpec((None, tc, tile_n), lambda b, n, c: (b, c, n)),
        out_shape=jax.ShapeDtypeStruct((B, C1 + C2, N), jnp.float32),
    )(x1, interp_t, stats)


def _stage(xyz1, xyz2, points1, points2_t, tile_n):
    B, N, _ = xyz1.shape
    C2 = points2_t.shape[1]
    xyz2t = jnp.transpose(xyz2, (0, 2, 1))
    idx, w = _knn_topk(xyz1, xyz2t, tile_n)
    interp_t, s1, s2 = _interp_matmul(idx, w, points2_t, 128, tile_n)
    stats = _mean_scale(s1, s2, float(B * N * C2))
    return _norm_concat(points1, interp_t, stats, min(N, 512))


def kernel(xyz_list_0, xyz_list_1, xyz_list_2, x_list_0, x_list_1, x_list_2):
    out1 = _stage(xyz_list_1, xyz_list_2, x_list_1, x_list_2, 256)
    out2 = _stage(xyz_list_0, xyz_list_1, x_list_0, out1, 256)
    return out2


# speedup vs baseline: 13.5030x; 1.1423x over previous
import functools

import jax
import jax.numpy as jnp
from jax import lax
from jax.experimental import pallas as pl
from jax.experimental.pallas import tpu as pltpu
from jax.experimental.pallas import tpu_sc as plsc

_K = 3
_BIG = 3.0e38
_NC, _NS = 2, 16
_NW = _NC * _NS
_QB = 16


def _knn_body(q_ref, p_ref, idx_ref, w_ref, *, S):
    q = q_ref[...]
    p = p_ref[...]
    qn = (q[:, 0:1] * q[:, 0:1] + q[:, 1:2] * q[:, 1:2]) + q[:, 2:3] * q[:, 2:3]
    pn = (p[0:1, :] * p[0:1, :] + p[1:2, :] * p[1:2, :]) + p[2:3, :] * p[2:3, :]
    cross = lax.dot_general(q.astype(jnp.bfloat16), p.astype(jnp.bfloat16),
                            (((1,), (0,)), ((), ())),
                            preferred_element_type=jnp.float32)
    d = -2.0 * cross + qn + pn
    iota = lax.broadcasted_iota(jnp.int32, d.shape, 1)
    idxs, vals = [], []
    for _ in range(_K):
        m = jnp.min(d, axis=1, keepdims=True)
        i = jnp.min(jnp.where(d == m, iota, S), axis=1, keepdims=True)
        idxs.append(i)
        vals.append(m)
        d = jnp.where(iota == i, _BIG, d)
    r = [1.0 / (v + 1e-8) for v in vals]
    norm = r[0] + r[1] + r[2]
    base = pl.program_id(0) * S
    idx_ref[...] = jnp.concatenate([i + base for i in idxs], axis=1)
    w_ref[...] = jnp.concatenate([ri / norm for ri in r], axis=1)


def _knn_topk(xyz1, xyz2t, tile_n):
    B, N, _ = xyz1.shape
    S = xyz2t.shape[2]
    grid = (B, N // tile_n)
    return pl.pallas_call(
        functools.partial(_knn_body, S=S),
        grid=grid,
        in_specs=[
            pl.BlockSpec((None, tile_n, 3), lambda b, n: (b, n, 0)),
            pl.BlockSpec((None, 3, S), lambda b, n: (b, 0, 0)),
        ],
        out_specs=[
            pl.BlockSpec((None, tile_n, _K), lambda b, n: (b, n, 0)),
            pl.BlockSpec((None, tile_n, _K), lambda b, n: (b, n, 0)),
        ],
        out_shape=[
            jax.ShapeDtypeStruct((B, N, _K), jnp.int32),
            jax.ShapeDtypeStruct((B, N, _K), jnp.float32),
        ],
    )(xyz1, xyz2t)


def _sc_gather_body(idx_hbm, w_hbm, table_hbm, out_hbm, stats_hbm,
                    idx_v, w_v, rows_v, out_v, stats_v, sem, *, C, qw):
    wid = lax.axis_index("s") * _NC + lax.axis_index("c")
    nb = qw // _QB

    def blk_body(blk, carry):
        acc_s, acc_q = carry
        base = wid * qw + blk * _QB
        pltpu.sync_copy(idx_hbm.at[pl.ds(base * 3, _QB * 3)], idx_v)
        pltpu.sync_copy(w_hbm.at[pl.ds(base * 3, _QB * 3)],
                        w_v.at[pl.ds(0, _QB * 3)])
        pltpu.async_copy(table_hbm.at[idx_v], rows_v, sem).wait()

        def q_body(q, carry2):
            a_s, a_q = carry2
            wtri = w_v[pl.ds(3 * q, 16)]
            w0 = wtri[0]
            w1 = wtri[1]
            w2 = wtri[2]
            for c in range(C // 16):
                r0 = rows_v[3 * q, pl.ds(c * 16, 16)]
                r1 = rows_v[3 * q + 1, pl.ds(c * 16, 16)]
                r2 = rows_v[3 * q + 2, pl.ds(c * 16, 16)]
                y = r0 * w0 + r1 * w1 + r2 * w2
                out_v[pl.ds(q * C + c * 16, 16)] = y
                a_s = a_s + y
                a_q = a_q + y * y
            return a_s, a_q

        acc = lax.fori_loop(0, _QB, q_body, (acc_s, acc_q))
        pltpu.sync_copy(out_v, out_hbm.at[pl.ds(base * C, _QB * C)])
        return acc

    z = jnp.zeros((16,), jnp.float32)
    acc_s, acc_q = lax.fori_loop(0, nb, blk_body, (z, z))
    stats_v[pl.ds(0, 16)] = acc_s
    stats_v[pl.ds(16, 16)] = acc_q
    pltpu.sync_copy(stats_v, stats_hbm.at[pl.ds(wid * 32, 32)])


def _sc_gather(idx, w, table):
    Q = idx.shape[0]
    C = table.shape[1]
    qw = Q // _NW
    mesh = plsc.VectorSubcoreMesh(core_axis_name="c", subcore_axis_name="s")
    out, stats = pl.kernel(
        functools.partial(_sc_gather_body, C=C, qw=qw),
        out_type=[
            jax.ShapeDtypeStruct((Q * C,), jnp.float32),
            jax.ShapeDtypeStruct((_NW * 32,), jnp.float32),
        ],
        mesh=mesh,
        scratch_types=[
            pltpu.VMEM((_QB * 3,), jnp.int32),
            pltpu.VMEM((_QB * 3 + 16,), jnp.float32),
            pltpu.VMEM((_QB * 3, C), jnp.float32),
            pltpu.VMEM((_QB * C,), jnp.float32),
            pltpu.VMEM((32,), jnp.float32),
            pltpu.SemaphoreType.DMA,
        ],
        name="sc_gather_interp",
    )(idx.reshape(-1), w.reshape(-1), table)
    st = stats.reshape(_NW, 32)
    return out.reshape(Q, C), jnp.sum(st[:, :16]), jnp.sum(st[:, 16:])


def _mean_scale(s1, s2, n):
    mean = s1 / n
    m2 = s2 - n * mean * mean
    std = jnp.sqrt(m2 / (n - 1))
    scale = 1.0 / (std + 1e-5)
    return jnp.stack([mean, scale]).astype(jnp.float32)


def _norm1_body(x1_ref, it_ref, st_ref, out_ref, *, C1):
    out_ref[:, :C1] = jnp.transpose(x1_ref[...])
    out_ref[:, C1:] = (it_ref[...] - st_ref[0]) * st_ref[1]


def _norm1(x1, interp, stats, tile_n):
    B, C1, N = x1.shape
    C2 = interp.shape[2]
    grid = (B, N // tile_n)
    return pl.pallas_call(
        functools.partial(_norm1_body, C1=C1),
        grid=grid,
        in_specs=[
            pl.BlockSpec((None, C1, tile_n), lambda b, n: (b, 0, n)),
            pl.BlockSpec((None, tile_n, C2), lambda b, n: (b, n, 0)),
            pl.BlockSpec(memory_space=pltpu.SMEM),
        ],
        out_specs=pl.BlockSpec((None, tile_n, C1 + C2), lambda b, n: (b, n, 0)),
        out_shape=jax.ShapeDtypeStruct((B, N, C1 + C2), jnp.float32),
    )(x1, interp, stats)


def _norm2_body(x0_ref, it_ref, st_ref, out_ref):
    c = pl.program_id(2)

    @pl.when(c == 0)
    def _copy():
        out_ref[...] = x0_ref[...]

    @pl.when(c > 0)
    def _norm():
        out_ref[...] = jnp.transpose((it_ref[...] - st_ref[0]) * st_ref[1])


def _norm2(x0, interp, stats, tile_n):
    B, C0, N = x0.shape
    C2 = interp.shape[2]
    ct = (C0 + C2) // C0
    grid = (B, N // tile_n, ct)
    return pl.pallas_call(
        _norm2_body,
        grid=grid,
        in_specs=[
            pl.BlockSpec((None, C0, tile_n), lambda b, n, c: (b, 0, n)),
            pl.BlockSpec((None, tile_n, C0),
                         lambda b, n, c: (b, n, jnp.maximum(c - 1, 0))),
            pl.BlockSpec(memory_space=pltpu.SMEM),
        ],
        out_specs=pl.BlockSpec((None, C0, tile_n), lambda b, n, c: (b, c, n)),
        out_shape=jax.ShapeDtypeStruct((B, C0 + C2, N), jnp.float32),
    )(x0, interp, stats)


def kernel(xyz_list_0, xyz_list_1, xyz_list_2, x_list_0, x_list_1, x_list_2):
    B, N1, _ = xyz_list_1.shape
    N2 = xyz_list_0.shape[1]
    S1 = xyz_list_2.shape[1]
    C2a = x_list_2.shape[1]

    idx1, w1 = _knn_topk(xyz_list_1, jnp.transpose(xyz_list_2, (0, 2, 1)), 256)
    idx2, w2 = _knn_topk(xyz_list_0, jnp.transpose(xyz_list_1, (0, 2, 1)), 256)

    table1 = jnp.transpose(x_list_2, (0, 2, 1)).reshape(B * S1, C2a)
    interp1, s1a, s2a = _sc_gather(idx1.reshape(-1, _K), w1.reshape(-1, _K),
                                   table1)
    st1 = _mean_scale(s1a, s2a, float(B * N1 * C2a))
    out1t = _norm1(x_list_1, interp1.reshape(B, N1, C2a), st1, 256)

    C2b = out1t.shape[2]
    interp2, s1b, s2b = _sc_gather(idx2.reshape(-1, _K), w2.reshape(-1, _K),
                                   out1t.reshape(B * N1, C2b))
    st2 = _mean_scale(s1b, s2b, float(B * N2 * C2b))
    return _norm2(x_list_0, interp2.reshape(B, N2, C2b), st2, 512)


# speedup vs baseline: 14.8414x; 1.0991x over previous
import functools

import jax
import jax.numpy as jnp
from jax import lax
from jax.experimental import pallas as pl
from jax.experimental.pallas import tpu as pltpu
from jax.experimental.pallas import tpu_sc as plsc

_K = 3
_BIG = 3.0e38
_NC, _NS = 2, 16
_NW = _NC * _NS
_QB = 16


def _knn_body(q_ref, p_ref, idx_ref, w_ref, *, S):
    q = q_ref[...]
    p = p_ref[...]
    qn = (q[:, 0:1] * q[:, 0:1] + q[:, 1:2] * q[:, 1:2]) + q[:, 2:3] * q[:, 2:3]
    pn = (p[0:1, :] * p[0:1, :] + p[1:2, :] * p[1:2, :]) + p[2:3, :] * p[2:3, :]
    cross = lax.dot_general(q.astype(jnp.bfloat16), p.astype(jnp.bfloat16),
                            (((1,), (0,)), ((), ())),
                            preferred_element_type=jnp.float32)
    d = -2.0 * cross + qn + pn
    iota = lax.broadcasted_iota(jnp.int32, d.shape, 1)
    idxs, vals = [], []
    for _ in range(_K):
        m = jnp.min(d, axis=1, keepdims=True)
        i = jnp.min(jnp.where(d == m, iota, S), axis=1, keepdims=True)
        idxs.append(i)
        vals.append(m)
        d = jnp.where(iota == i, _BIG, d)
    r = [1.0 / (v + 1e-8) for v in vals]
    norm = r[0] + r[1] + r[2]
    base = pl.program_id(0) * S
    idx_ref[...] = jnp.concatenate([i + base for i in idxs], axis=1)
    w_ref[...] = jnp.concatenate([ri / norm for ri in r], axis=1)


def _knn_topk(xyz1, xyz2t, tile_n):
    B, N, _ = xyz1.shape
    S = xyz2t.shape[2]
    grid = (B, N // tile_n)
    return pl.pallas_call(
        functools.partial(_knn_body, S=S),
        grid=grid,
        in_specs=[
            pl.BlockSpec((None, tile_n, 3), lambda b, n: (b, n, 0)),
            pl.BlockSpec((None, 3, S), lambda b, n: (b, 0, 0)),
        ],
        out_specs=[
            pl.BlockSpec((None, tile_n, _K), lambda b, n: (b, n, 0)),
            pl.BlockSpec((None, tile_n, _K), lambda b, n: (b, n, 0)),
        ],
        out_shape=[
            jax.ShapeDtypeStruct((B, N, _K), jnp.int32),
            jax.ShapeDtypeStruct((B, N, _K), jnp.float32),
        ],
    )(xyz1, xyz2t)


def _sc_gather_body(idx_hbm, w_hbm, table_hbm, out_hbm, stats_hbm,
                    idx_v0, idx_v1, w_v, rows_v0, rows_v1, out_v, stats_v,
                    sem0, sem1, *, C, qw):
    wid = lax.axis_index("s") * _NC + lax.axis_index("c")
    nb = qw // _QB
    base0 = wid * qw
    idx_vs = (idx_v0, idx_v1)
    rows_vs = (rows_v0, rows_v1)
    sems = (sem0, sem1)

    def issue(blk, par):
        base = base0 + blk * _QB
        pltpu.sync_copy(idx_hbm.at[pl.ds(base * 3, _QB * 3)], idx_vs[par])
        pltpu.async_copy(table_hbm.at[idx_vs[par]], rows_vs[par], sems[par])

    def compute(blk, par, carry):
        base = base0 + blk * _QB
        pltpu.sync_copy(w_hbm.at[pl.ds(base * 3, _QB * 3)],
                        w_v.at[pl.ds(0, _QB * 3)])
        pltpu.make_async_copy(table_hbm.at[idx_vs[par]], rows_vs[par],
                              sems[par]).wait()
        rows_v = rows_vs[par]

        def q_body(q, carry2):
            a_s, a_q = carry2
            wtri = w_v[pl.ds(3 * q, 16)]
            w0 = wtri[0]
            w1 = wtri[1]
            w2 = wtri[2]
            for c in range(C // 16):
                r0 = rows_v[3 * q, pl.ds(c * 16, 16)]
                r1 = rows_v[3 * q + 1, pl.ds(c * 16, 16)]
                r2 = rows_v[3 * q + 2, pl.ds(c * 16, 16)]
                y = r0 * w0 + r1 * w1 + r2 * w2
                out_v[pl.ds(q * C + c * 16, 16)] = y
                a_s = a_s + y
                a_q = a_q + y * y
            return a_s, a_q

        carry = lax.fori_loop(0, _QB, q_body, carry)
        pltpu.sync_copy(out_v, out_hbm.at[pl.ds(base * C, _QB * C)])
        return carry

    issue(0, 0)

    def pair_body(i, carry):
        blk = 2 * i
        issue(blk + 1, 1)
        carry = compute(blk, 0, carry)

        @pl.when(i < nb // 2 - 1)
        def _():
            issue(blk + 2, 0)

        return compute(blk + 1, 1, carry)

    z = jnp.zeros((16,), jnp.float32)
    acc_s, acc_q = lax.fori_loop(0, nb // 2, pair_body, (z, z))
    stats_v[pl.ds(0, 16)] = acc_s
    stats_v[pl.ds(16, 16)] = acc_q
    pltpu.sync_copy(stats_v, stats_hbm.at[pl.ds(wid * 32, 32)])


def _sc_gather(idx, w, table):
    Q = idx.shape[0]
    C = table.shape[1]
    qw = Q // _NW
    mesh = plsc.VectorSubcoreMesh(core_axis_name="c", subcore_axis_name="s")
    out, stats = pl.kernel(
        functools.partial(_sc_gather_body, C=C, qw=qw),
        out_type=[
            jax.ShapeDtypeStruct((Q * C,), jnp.float32),
            jax.ShapeDtypeStruct((_NW * 32,), jnp.float32),
        ],
        mesh=mesh,
        scratch_types=[
            pltpu.VMEM((_QB * 3,), jnp.int32),
            pltpu.VMEM((_QB * 3,), jnp.int32),
            pltpu.VMEM((_QB * 3 + 16,), jnp.float32),
            pltpu.VMEM((_QB * 3, C), jnp.float32),
            pltpu.VMEM((_QB * 3, C), jnp.float32),
            pltpu.VMEM((_QB * C,), jnp.float32),
            pltpu.VMEM((32,), jnp.float32),
            pltpu.SemaphoreType.DMA,
            pltpu.SemaphoreType.DMA,
        ],
        name="sc_gather_interp",
    )(idx.reshape(-1), w.reshape(-1), table)
    st = stats.reshape(_NW, 32)
    return out.reshape(Q, C), jnp.sum(st[:, :16]), jnp.sum(st[:, 16:])


def _mean_scale(s1, s2, n):
    mean = s1 / n
    m2 = s2 - n * mean * mean
    std = jnp.sqrt(m2 / (n - 1))
    scale = 1.0 / (std + 1e-5)
    return jnp.stack([mean, scale]).astype(jnp.float32)


def _norm1_body(x1_ref, it_ref, st_ref, out_ref, *, C1):
    out_ref[:, :C1] = jnp.transpose(x1_ref[...])
    out_ref[:, C1:] = (it_ref[...] - st_ref[0]) * st_ref[1]


def _norm1(x1, interp, stats, tile_n):
    B, C1, N = x1.shape
    C2 = interp.shape[2]
    grid = (B, N // tile_n)
    return pl.pallas_call(
        functools.partial(_norm1_body, C1=C1),
        grid=grid,
        in_specs=[
            pl.BlockSpec((None, C1, tile_n), lambda b, n: (b, 0, n)),
            pl.BlockSpec((None, tile_n, C2), lambda b, n: (b, n, 0)),
            pl.BlockSpec(memory_space=pltpu.SMEM),
        ],
        out_specs=pl.BlockSpec((None, tile_n, C1 + C2), lambda b, n: (b, n, 0)),
        out_shape=jax.ShapeDtypeStruct((B, N, C1 + C2), jnp.float32),
    )(x1, interp, stats)


def _norm2_body(x0_ref, it_ref, st_ref, out_ref):
    c = pl.program_id(2)

    @pl.when(c == 0)
    def _copy():
        out_ref[...] = x0_ref[...]

    @pl.when(c > 0)
    def _norm():
        out_ref[...] = jnp.transpose((it_ref[...] - st_ref[0]) * st_ref[1])


def _norm2(x0, interp, stats, tile_n):
    B, C0, N = x0.shape
    C2 = interp.shape[2]
    ct = (C0 + C2) // C0
    grid = (B, N // tile_n, ct)
    return pl.pallas_call(
        _norm2_body,
        grid=grid,
        in_specs=[
            pl.BlockSpec((None, C0, tile_n), lambda b, n, c: (b, 0, n)),
            pl.BlockSpec((None, tile_n, C0),
                         lambda b, n, c: (b, n, jnp.maximum(c - 1, 0))),
            pl.BlockSpec(memory_space=pltpu.SMEM),
        ],
        out_specs=pl.BlockSpec((None, C0, tile_n), lambda b, n, c: (b, c, n)),
        out_shape=jax.ShapeDtypeStruct((B, C0 + C2, N), jnp.float32),
    )(x0, interp, stats)


def kernel(xyz_list_0, xyz_list_1, xyz_list_2, x_list_0, x_list_1, x_list_2):
    B, N1, _ = xyz_list_1.shape
    N2 = xyz_list_0.shape[1]
    S1 = xyz_list_2.shape[1]
    C2a = x_list_2.shape[1]

    idx1, w1 = _knn_topk(xyz_list_1, jnp.transpose(xyz_list_2, (0, 2, 1)), 256)
    idx2, w2 = _knn_topk(xyz_list_0, jnp.transpose(xyz_list_1, (0, 2, 1)), 256)

    table1 = jnp.transpose(x_list_2, (0, 2, 1)).reshape(B * S1, C2a)
    interp1, s1a, s2a = _sc_gather(idx1.reshape(-1, _K), w1.reshape(-1, _K),
                                   table1)
    st1 = _mean_scale(s1a, s2a, float(B * N1 * C2a))
    out1t = _norm1(x_list_1, interp1.reshape(B, N1, C2a), st1, 256)

    C2b = out1t.shape[2]
    interp2, s1b, s2b = _sc_gather(idx2.reshape(-1, _K), w2.reshape(-1, _K),
                                   out1t.reshape(B * N1, C2b))
    st2 = _mean_scale(s1b, s2b, float(B * N2 * C2b))
    return _norm2(x_list_0, interp2.reshape(B, N2, C2b), st2, 512)


# speedup vs baseline: 15.9471x; 1.0745x over previous
import functools

import jax
import jax.numpy as jnp
from jax import lax
from jax.experimental import pallas as pl
from jax.experimental.pallas import tpu as pltpu
from jax.experimental.pallas import tpu_sc as plsc

_K = 3
_BIG = 3.0e38
_NC, _NS = 2, 16
_NW = _NC * _NS
_QB = 16


def _knn_body(q_ref, p_ref, idx_ref, w_ref, *, S):
    q = q_ref[...]
    p = p_ref[...]
    qn = (q[:, 0:1] * q[:, 0:1] + q[:, 1:2] * q[:, 1:2]) + q[:, 2:3] * q[:, 2:3]
    pn = (p[0:1, :] * p[0:1, :] + p[1:2, :] * p[1:2, :]) + p[2:3, :] * p[2:3, :]
    cross = lax.dot_general(q.astype(jnp.bfloat16), p.astype(jnp.bfloat16),
                            (((1,), (0,)), ((), ())),
                            preferred_element_type=jnp.float32)
    d = -2.0 * cross + qn + pn
    iota = lax.broadcasted_iota(jnp.int32, d.shape, 1)
    idxs, vals = [], []
    for _ in range(_K):
        m = jnp.min(d, axis=1, keepdims=True)
        i = jnp.min(jnp.where(d == m, iota, S), axis=1, keepdims=True)
        idxs.append(i)
        vals.append(m)
        d = jnp.where(iota == i, _BIG, d)
    r = [1.0 / (v + 1e-8) for v in vals]
    norm = r[0] + r[1] + r[2]
    base = pl.program_id(0) * S
    idx_ref[...] = jnp.concatenate([i + base for i in idxs], axis=1)
    w_ref[...] = jnp.concatenate([ri / norm for ri in r], axis=1)


def _knn_topk(xyz1, xyz2t, tile_n):
    B, N, _ = xyz1.shape
    S = xyz2t.shape[2]
    grid = (B, N // tile_n)
    return pl.pallas_call(
        functools.partial(_knn_body, S=S),
        grid=grid,
        in_specs=[
            pl.BlockSpec((None, tile_n, 3), lambda b, n: (b, n, 0)),
            pl.BlockSpec((None, 3, S), lambda b, n: (b, 0, 0)),
        ],
        out_specs=[
            pl.BlockSpec((None, tile_n, _K), lambda b, n: (b, n, 0)),
            pl.BlockSpec((None, tile_n, _K), lambda b, n: (b, n, 0)),
        ],
        out_shape=[
            jax.ShapeDtypeStruct((B, N, _K), jnp.int32),
            jax.ShapeDtypeStruct((B, N, _K), jnp.float32),
        ],
    )(xyz1, xyz2t)


def _sc_gather_body(idx_hbm, w_hbm, table_hbm, out_hbm, stats_hbm,
                    idxall_v, wall_v, rows_v0, rows_v1, out_v0, out_v1,
                    stats_v, semg0, semg1, semo0, semo1, *, C, qw):
    wid = lax.axis_index("s") * _NC + lax.axis_index("c")
    nb = qw // _QB
    base0 = wid * qw
    rows_vs = (rows_v0, rows_v1)
    out_vs = (out_v0, out_v1)
    semgs = (semg0, semg1)
    semos = (semo0, semo1)

    pltpu.sync_copy(idx_hbm.at[pl.ds(wid * nb, nb)], idxall_v)
    pltpu.sync_copy(w_hbm.at[pl.ds(base0 * 3, qw * 3)],
                    wall_v.at[pl.ds(0, qw * 3)])

    def issue(blk, par):
        pltpu.async_copy(table_hbm.at[idxall_v.at[blk]], rows_vs[par],
                         semgs[par])

    def compute(blk, par, carry):
        base = base0 + blk * _QB
        pltpu.make_async_copy(table_hbm.at[idxall_v.at[blk]], rows_vs[par],
                              semgs[par]).wait()
        rows_v = rows_vs[par]
        out_v = out_vs[par]

        @pl.when(blk >= 2)
        def _():
            pltpu.make_async_copy(out_v, out_hbm.at[pl.ds(base * C, _QB * C)],
                                  semos[par]).wait()

        def q_body(q, carry2):
            a_s, a_q = carry2
            wtri = wall_v[pl.ds(blk * (3 * _QB) + 3 * q, 16)]
            w0 = wtri[0]
            w1 = wtri[1]
            w2 = wtri[2]
            for c in range(C // 16):
                r0 = rows_v[3 * q, pl.ds(c * 16, 16)]
                r1 = rows_v[3 * q + 1, pl.ds(c * 16, 16)]
                r2 = rows_v[3 * q + 2, pl.ds(c * 16, 16)]
                y = r0 * w0 + r1 * w1 + r2 * w2
                out_v[pl.ds(q * C + c * 16, 16)] = y
                a_s = a_s + y
                a_q = a_q + y * y
            return a_s, a_q

        carry = lax.fori_loop(0, _QB, q_body, carry)
        pltpu.async_copy(out_v, out_hbm.at[pl.ds(base * C, _QB * C)],
                         semos[par])
        return carry

    issue(0, 0)

    def pair_body(i, carry):
        blk = 2 * i
        issue(blk + 1, 1)
        carry = compute(blk, 0, carry)

        @pl.when(i < nb // 2 - 1)
        def _():
            issue(blk + 2, 0)

        return compute(blk + 1, 1, carry)

    z = jnp.zeros((16,), jnp.float32)
    acc_s, acc_q = lax.fori_loop(0, nb // 2, pair_body, (z, z))
    for par in (0, 1):
        pltpu.make_async_copy(out_vs[par],
                              out_hbm.at[pl.ds(base0 * C, _QB * C)],
                              semos[par]).wait()
    stats_v[pl.ds(0, 16)] = acc_s
    stats_v[pl.ds(16, 16)] = acc_q
    pltpu.sync_copy(stats_v, stats_hbm.at[pl.ds(wid * 32, 32)])


def _sc_gather(idx, w, table):
    Q = idx.shape[0]
    C = table.shape[1]
    qw = Q // _NW
    nb = qw // _QB
    mesh = plsc.VectorSubcoreMesh(core_axis_name="c", subcore_axis_name="s")
    out, stats = pl.kernel(
        functools.partial(_sc_gather_body, C=C, qw=qw),
        out_type=[
            jax.ShapeDtypeStruct((Q * C,), jnp.float32),
            jax.ShapeDtypeStruct((_NW * 32,), jnp.float32),
        ],
        mesh=mesh,
        scratch_types=[
            pltpu.VMEM((nb, _QB * 3), jnp.int32),
            pltpu.VMEM((qw * 3 + 16,), jnp.float32),
            pltpu.VMEM((_QB * 3, C), jnp.float32),
            pltpu.VMEM((_QB * 3, C), jnp.float32),
            pltpu.VMEM((_QB * C,), jnp.float32),
            pltpu.VMEM((_QB * C,), jnp.float32),
            pltpu.VMEM((32,), jnp.float32),
            pltpu.SemaphoreType.DMA,
            pltpu.SemaphoreType.DMA,
            pltpu.SemaphoreType.DMA,
            pltpu.SemaphoreType.DMA,
        ],
        name="sc_gather_interp",
    )(idx.reshape(-1, _QB * 3), w.reshape(-1), table)
    st = stats.reshape(_NW, 32)
    return out.reshape(Q, C), jnp.sum(st[:, :16]), jnp.sum(st[:, 16:])


def _mean_scale(s1, s2, n):
    mean = s1 / n
    m2 = s2 - n * mean * mean
    std = jnp.sqrt(m2 / (n - 1))
    scale = 1.0 / (std + 1e-5)
    return jnp.stack([mean, scale]).astype(jnp.float32)


def _norm1_body(x1_ref, it_ref, st_ref, out_ref, *, C1):
    out_ref[:, :C1] = jnp.transpose(x1_ref[...])
    out_ref[:, C1:] = (it_ref[...] - st_ref[0]) * st_ref[1]


def _norm1(x1, interp, stats, tile_n):
    B, C1, N = x1.shape
    C2 = interp.shape[2]
    grid = (B, N // tile_n)
    return pl.pallas_call(
        functools.partial(_norm1_body, C1=C1),
        grid=grid,
        in_specs=[
            pl.BlockSpec((None, C1, tile_n), lambda b, n: (b, 0, n)),
            pl.BlockSpec((None, tile_n, C2), lambda b, n: (b, n, 0)),
            pl.BlockSpec(memory_space=pltpu.SMEM),
        ],
        out_specs=pl.BlockSpec((None, tile_n, C1 + C2), lambda b, n: (b, n, 0)),
        out_shape=jax.ShapeDtypeStruct((B, N, C1 + C2), jnp.float32),
    )(x1, interp, stats)


def _norm2_body(x0_ref, it_ref, st_ref, out_ref):
    c = pl.program_id(2)

    @pl.when(c == 0)
    def _copy():
        out_ref[...] = x0_ref[...]

    @pl.when(c > 0)
    def _norm():
        out_ref[...] = jnp.transpose((it_ref[...] - st_ref[0]) * st_ref[1])


def _norm2(x0, interp, stats, tile_n):
    B, C0, N = x0.shape
    C2 = interp.shape[2]
    ct = (C0 + C2) // C0
    grid = (B, N // tile_n, ct)
    return pl.pallas_call(
        _norm2_body,
        grid=grid,
        in_specs=[
            pl.BlockSpec((None, C0, tile_n), lambda b, n, c: (b, 0, n)),
            pl.BlockSpec((None, tile_n, C0),
                         lambda b, n, c: (b, n, jnp.maximum(c - 1, 0))),
            pl.BlockSpec(memory_space=pltpu.SMEM),
        ],
        out_specs=pl.BlockSpec((None, C0, tile_n), lambda b, n, c: (b, c, n)),
        out_shape=jax.ShapeDtypeStruct((B, C0 + C2, N), jnp.float32),
    )(x0, interp, stats)


def kernel(xyz_list_0, xyz_list_1, xyz_list_2, x_list_0, x_list_1, x_list_2):
    B, N1, _ = xyz_list_1.shape
    N2 = xyz_list_0.shape[1]
    S1 = xyz_list_2.shape[1]
    C2a = x_list_2.shape[1]

    idx1, w1 = _knn_topk(xyz_list_1, jnp.transpose(xyz_list_2, (0, 2, 1)), 256)
    idx2, w2 = _knn_topk(xyz_list_0, jnp.transpose(xyz_list_1, (0, 2, 1)), 256)

    table1 = jnp.transpose(x_list_2, (0, 2, 1)).reshape(B * S1, C2a)
    interp1, s1a, s2a = _sc_gather(idx1.reshape(-1, _K), w1.reshape(-1, _K),
                                   table1)
    st1 = _mean_scale(s1a, s2a, float(B * N1 * C2a))
    out1t = _norm1(x_list_1, interp1.reshape(B, N1, C2a), st1, 256)

    C2b = out1t.shape[2]
    interp2, s1b, s2b = _sc_gather(idx2.reshape(-1, _K), w2.reshape(-1, _K),
                                   out1t.reshape(B * N1, C2b))
    st2 = _mean_scale(s1b, s2b, float(B * N2 * C2b))
    return _norm2(x_list_0, interp2.reshape(B, N2, C2b), st2, 512)


# speedup vs baseline: 16.7923x; 1.0530x over previous
import functools

import jax
import jax.numpy as jnp
from jax import lax
from jax.experimental import pallas as pl
from jax.experimental.pallas import tpu as pltpu
from jax.experimental.pallas import tpu_sc as plsc

_K = 3
_BIG = 3.0e38
_NC, _NS = 2, 16
_NW = _NC * _NS
_QB = 16


def _knn_body(q_ref, p_ref, idx_ref, w_ref, *, S):
    q = q_ref[...]
    p = p_ref[...]
    qn = (q[:, 0:1] * q[:, 0:1] + q[:, 1:2] * q[:, 1:2]) + q[:, 2:3] * q[:, 2:3]
    pn = (p[0:1, :] * p[0:1, :] + p[1:2, :] * p[1:2, :]) + p[2:3, :] * p[2:3, :]
    cross = lax.dot_general(q.astype(jnp.bfloat16), p.astype(jnp.bfloat16),
                            (((1,), (0,)), ((), ())),
                            preferred_element_type=jnp.float32)
    d = -2.0 * cross + qn + pn
    iota = lax.broadcasted_iota(jnp.int32, d.shape, 1).astype(jnp.float32)
    idxs, vals = [], []
    for _ in range(_K):
        m = jnp.min(d, axis=1, keepdims=True)
        i = jnp.min(jnp.where(d == m, iota, _BIG), axis=1, keepdims=True)
        idxs.append(i)
        vals.append(m)
        d = jnp.where(iota == i, _BIG, d)
    r = [1.0 / (v + 1e-8) for v in vals]
    norm = r[0] + r[1] + r[2]
    base = pl.program_id(0) * S
    idx_ref[...] = jnp.concatenate(
        [i.astype(jnp.int32) + base for i in idxs], axis=1)
    w_ref[...] = jnp.concatenate([ri / norm for ri in r], axis=1)


def _knn_topk(xyz1, xyz2t, tile_n):
    B, N, _ = xyz1.shape
    S = xyz2t.shape[2]
    grid = (B, N // tile_n)
    return pl.pallas_call(
        functools.partial(_knn_body, S=S),
        grid=grid,
        in_specs=[
            pl.BlockSpec((None, tile_n, 3), lambda b, n: (b, n, 0)),
            pl.BlockSpec((None, 3, S), lambda b, n: (b, 0, 0)),
        ],
        out_specs=[
            pl.BlockSpec((None, tile_n, _K), lambda b, n: (b, n, 0)),
            pl.BlockSpec((None, tile_n, _K), lambda b, n: (b, n, 0)),
        ],
        out_shape=[
            jax.ShapeDtypeStruct((B, N, _K), jnp.int32),
            jax.ShapeDtypeStruct((B, N, _K), jnp.float32),
        ],
    )(xyz1, xyz2t)


def _sc_gather_body(idx_hbm, w_hbm, table_hbm, out_hbm, stats_hbm,
                    idxall_v, wall_v, rows_v0, rows_v1, out_v0, out_v1,
                    stats_v, semg0, semg1, semo0, semo1, *, C, qw):
    wid = lax.axis_index("s") * _NC + lax.axis_index("c")
    nb = qw // _QB
    base0 = wid * qw
    rows_vs = (rows_v0, rows_v1)
    out_vs = (out_v0, out_v1)
    semgs = (semg0, semg1)
    semos = (semo0, semo1)

    pltpu.sync_copy(idx_hbm.at[pl.ds(wid * nb, nb)], idxall_v)
    pltpu.sync_copy(w_hbm.at[pl.ds(base0 * 3, qw * 3)],
                    wall_v.at[pl.ds(0, qw * 3)])

    def issue(blk, par):
        pltpu.async_copy(table_hbm.at[idxall_v.at[blk]], rows_vs[par],
                         semgs[par])

    def compute(blk, par, carry):
        base = base0 + blk * _QB
        pltpu.make_async_copy(table_hbm.at[idxall_v.at[blk]], rows_vs[par],
                              semgs[par]).wait()
        rows_v = rows_vs[par]
        out_v = out_vs[par]

        @pl.when(blk >= 2)
        def _():
            pltpu.make_async_copy(out_v, out_hbm.at[pl.ds(base * C, _QB * C)],
                                  semos[par]).wait()

        def q_body(q, carry2):
            a_s, a_q = carry2
            wtri = wall_v[pl.ds(blk * (3 * _QB) + 3 * q, 16)]
            w0 = wtri[0]
            w1 = wtri[1]
            w2 = wtri[2]
            for c in range(C // 16):
                r0 = rows_v[3 * q, pl.ds(c * 16, 16)]
                r1 = rows_v[3 * q + 1, pl.ds(c * 16, 16)]
                r2 = rows_v[3 * q + 2, pl.ds(c * 16, 16)]
                y = r0 * w0 + r1 * w1 + r2 * w2
                out_v[pl.ds(q * C + c * 16, 16)] = y
                a_s = a_s + y
                a_q = a_q + y * y
            return a_s, a_q

        carry = lax.fori_loop(0, _QB, q_body, carry)
        pltpu.async_copy(out_v, out_hbm.at[pl.ds(base * C, _QB * C)],
                         semos[par])
        return carry

    issue(0, 0)

    def pair_body(i, carry):
        blk = 2 * i
        issue(blk + 1, 1)
        carry = compute(blk, 0, carry)

        @pl.when(i < nb // 2 - 1)
        def _():
            issue(blk + 2, 0)

        return compute(blk + 1, 1, carry)

    z = jnp.zeros((16,), jnp.float32)
    acc_s, acc_q = lax.fori_loop(0, nb // 2, pair_body, (z, z))
    for par in (0, 1):
        pltpu.make_async_copy(out_vs[par],
                              out_hbm.at[pl.ds(base0 * C, _QB * C)],
                              semos[par]).wait()
    stats_v[pl.ds(0, 16)] = acc_s
    stats_v[pl.ds(16, 16)] = acc_q
    pltpu.sync_copy(stats_v, stats_hbm.at[pl.ds(wid * 32, 32)])


def _sc_gather(idx, w, table):
    Q = idx.shape[0]
    C = table.shape[1]
    qw = Q // _NW
    nb = qw // _QB
    mesh = plsc.VectorSubcoreMesh(core_axis_name="c", subcore_axis_name="s")
    out, stats = pl.kernel(
        functools.partial(_sc_gather_body, C=C, qw=qw),
        out_type=[
            jax.ShapeDtypeStruct((Q * C,), jnp.float32),
            jax.ShapeDtypeStruct((_NW * 32,), jnp.float32),
        ],
        mesh=mesh,
        scratch_types=[
            pltpu.VMEM((nb, _QB * 3), jnp.int32),
            pltpu.VMEM((qw * 3 + 16,), jnp.float32),
            pltpu.VMEM((_QB * 3, C), jnp.float32),
            pltpu.VMEM((_QB * 3, C), jnp.float32),
            pltpu.VMEM((_QB * C,), jnp.float32),
            pltpu.VMEM((_QB * C,), jnp.float32),
            pltpu.VMEM((32,), jnp.float32),
            pltpu.SemaphoreType.DMA,
            pltpu.SemaphoreType.DMA,
            pltpu.SemaphoreType.DMA,
            pltpu.SemaphoreType.DMA,
        ],
        name="sc_gather_interp",
    )(idx.reshape(-1, _QB * 3), w.reshape(-1), table)
    st = stats.reshape(_NW, 32)
    return out.reshape(Q, C), jnp.sum(st[:, :16]), jnp.sum(st[:, 16:])


def _mean_scale(s1, s2, n):
    mean = s1 / n
    m2 = s2 - n * mean * mean
    std = jnp.sqrt(m2 / (n - 1))
    scale = 1.0 / (std + 1e-5)
    return jnp.stack([mean, scale]).astype(jnp.float32)


def _norm1_body(x1_ref, it_ref, st_ref, out_ref, *, C1):
    out_ref[:, :C1] = jnp.transpose(x1_ref[...])
    out_ref[:, C1:] = (it_ref[...] - st_ref[0]) * st_ref[1]


def _norm1(x1, interp, stats, tile_n):
    B, C1, N = x1.shape
    C2 = interp.shape[2]
    grid = (B, N // tile_n)
    return pl.pallas_call(
        functools.partial(_norm1_body, C1=C1),
        grid=grid,
        in_specs=[
            pl.BlockSpec((None, C1, tile_n), lambda b, n: (b, 0, n)),
            pl.BlockSpec((None, tile_n, C2), lambda b, n: (b, n, 0)),
            pl.BlockSpec(memory_space=pltpu.SMEM),
        ],
        out_specs=pl.BlockSpec((None, tile_n, C1 + C2), lambda b, n: (b, n, 0)),
        out_shape=jax.ShapeDtypeStruct((B, N, C1 + C2), jnp.float32),
    )(x1, interp, stats)


def _norm2_body(x0_ref, it_ref, st_ref, out_ref):
    c = pl.program_id(2)

    @pl.when(c == 0)
    def _copy():
        out_ref[...] = x0_ref[...]

    @pl.when(c > 0)
    def _norm():
        out_ref[...] = jnp.transpose((it_ref[...] - st_ref[0]) * st_ref[1])


def _norm2(x0, interp, stats, tile_n):
    B, C0, N = x0.shape
    C2 = interp.shape[2]
    ct = (C0 + C2) // C0
    grid = (B, N // tile_n, ct)
    return pl.pallas_call(
        _norm2_body,
        grid=grid,
        in_specs=[
            pl.BlockSpec((None, C0, tile_n), lambda b, n, c: (b, 0, n)),
            pl.BlockSpec((None, tile_n, C0),
                         lambda b, n, c: (b, n, jnp.maximum(c - 1, 0))),
            pl.BlockSpec(memory_space=pltpu.SMEM),
        ],
        out_specs=pl.BlockSpec((None, C0, tile_n), lambda b, n, c: (b, c, n)),
        out_shape=jax.ShapeDtypeStruct((B, C0 + C2, N), jnp.float32),
    )(x0, interp, stats)


def kernel(xyz_list_0, xyz_list_1, xyz_list_2, x_list_0, x_list_1, x_list_2):
    B, N1, _ = xyz_list_1.shape
    N2 = xyz_list_0.shape[1]
    S1 = xyz_list_2.shape[1]
    C2a = x_list_2.shape[1]

    idx1, w1 = _knn_topk(xyz_list_1, jnp.transpose(xyz_list_2, (0, 2, 1)), 256)
    idx2, w2 = _knn_topk(xyz_list_0, jnp.transpose(xyz_list_1, (0, 2, 1)), 256)

    table1 = jnp.transpose(x_list_2, (0, 2, 1)).reshape(B * S1, C2a)
    interp1, s1a, s2a = _sc_gather(idx1.reshape(-1, _K), w1.reshape(-1, _K),
                                   table1)
    st1 = _mean_scale(s1a, s2a, float(B * N1 * C2a))
    out1t = _norm1(x_list_1, interp1.reshape(B, N1, C2a), st1, 256)

    C2b = out1t.shape[2]
    interp2, s1b, s2b = _sc_gather(idx2.reshape(-1, _K), w2.reshape(-1, _K),
                                   out1t.reshape(B * N1, C2b))
    st2 = _mean_scale(s1b, s2b, float(B * N2 * C2b))
    return _norm2(x_list_0, interp2.reshape(B, N2, C2b), st2, 512)


# speedup vs baseline: 17.1335x; 1.0203x over previous
import functools

import jax
import jax.numpy as jnp
from jax import lax
from jax.experimental import pallas as pl
from jax.experimental.pallas import tpu as pltpu
from jax.experimental.pallas import tpu_sc as plsc

_K = 3
_BIG = 3.0e38
_NC, _NS = 2, 16
_NW = _NC * _NS
_QB = 16


def _knn_body(q_ref, p_ref, idx_ref, w_ref, *, S):
    q = q_ref[...]
    p = p_ref[...]
    qn = (q[:, 0:1] * q[:, 0:1] + q[:, 1:2] * q[:, 1:2]) + q[:, 2:3] * q[:, 2:3]
    pn = (p[0:1, :] * p[0:1, :] + p[1:2, :] * p[1:2, :]) + p[2:3, :] * p[2:3, :]
    cross = lax.dot_general(q.astype(jnp.bfloat16), p.astype(jnp.bfloat16),
                            (((1,), (0,)), ((), ())),
                            preferred_element_type=jnp.float32)
    d = -2.0 * cross + qn + pn
    iota = lax.broadcasted_iota(jnp.int32, d.shape, 1).astype(jnp.float32)
    idxs, vals = [], []
    for _ in range(_K):
        m = jnp.min(d, axis=1, keepdims=True)
        i = jnp.min(jnp.where(d == m, iota, _BIG), axis=1, keepdims=True)
        idxs.append(i)
        vals.append(m)
        d = jnp.where(iota == i, _BIG, d)
    r = [1.0 / (v + 1e-8) for v in vals]
    norm = r[0] + r[1] + r[2]
    base = pl.program_id(0) * S
    idx_ref[...] = jnp.concatenate(
        [i.astype(jnp.int32) + base for i in idxs], axis=1)
    w_ref[...] = jnp.concatenate([ri / norm for ri in r], axis=1)


def _knn_topk(xyz1, xyz2t, tile_n):
    B, N, _ = xyz1.shape
    S = xyz2t.shape[2]
    grid = (B, N // tile_n)
    return pl.pallas_call(
        functools.partial(_knn_body, S=S),
        grid=grid,
        in_specs=[
            pl.BlockSpec((None, tile_n, 3), lambda b, n: (b, n, 0)),
            pl.BlockSpec((None, 3, S), lambda b, n: (b, 0, 0)),
        ],
        out_specs=[
            pl.BlockSpec((None, tile_n, _K), lambda b, n: (b, n, 0)),
            pl.BlockSpec((None, tile_n, _K), lambda b, n: (b, n, 0)),
        ],
        out_shape=[
            jax.ShapeDtypeStruct((B, N, _K), jnp.int32),
            jax.ShapeDtypeStruct((B, N, _K), jnp.float32),
        ],
    )(xyz1, xyz2t)


def _sc_gather_body(idx_hbm, w_hbm, table_hbm, out_hbm, stats_hbm,
                    idxall_v, wall_v, rows_v0, rows_v1, out_v0, out_v1,
                    stats_v, semg0, semg1, semo0, semo1, *, C, qw):
    wid = lax.axis_index("s") * _NC + lax.axis_index("c")
    nb = qw // _QB
    base0 = wid * qw
    rows_vs = (rows_v0, rows_v1)
    out_vs = (out_v0, out_v1)
    semgs = (semg0, semg1)
    semos = (semo0, semo1)

    pltpu.sync_copy(idx_hbm.at[pl.ds(wid * nb, nb)], idxall_v)
    pltpu.sync_copy(w_hbm.at[pl.ds(base0 * 3, qw * 3)],
                    wall_v.at[pl.ds(0, qw * 3)])

    def issue(blk, par):
        pltpu.async_copy(table_hbm.at[idxall_v.at[blk]], rows_vs[par],
                         semgs[par])

    def compute(blk, par, carry):
        base = base0 + blk * _QB
        pltpu.make_async_copy(table_hbm.at[idxall_v.at[blk]], rows_vs[par],
                              semgs[par]).wait()
        rows_v = rows_vs[par]
        out_v = out_vs[par]

        @pl.when(blk >= 2)
        def _():
            pltpu.make_async_copy(out_v, out_hbm.at[pl.ds(base * C, _QB * C)],
                                  semos[par]).wait()

        def q_body(q, carry2):
            a_s, a_q = carry2
            wtri = wall_v[pl.ds(blk * (3 * _QB) + 3 * q, 16)]
            w0 = wtri[0]
            w1 = wtri[1]
            w2 = wtri[2]
            for c in range(C // 16):
                r0 = rows_v[3 * q, pl.ds(c * 16, 16)]
                r1 = rows_v[3 * q + 1, pl.ds(c * 16, 16)]
                r2 = rows_v[3 * q + 2, pl.ds(c * 16, 16)]
                y = r0 * w0 + r1 * w1 + r2 * w2
                out_v[pl.ds(q * C + c * 16, 16)] = y
                a_s = a_s + y
                a_q = a_q + y * y
            return a_s, a_q

        carry = lax.fori_loop(0, _QB, q_body, carry)
        pltpu.async_copy(out_v, out_hbm.at[pl.ds(base * C, _QB * C)],
                         semos[par])
        return carry

    issue(0, 0)

    def pair_body(i, carry):
        blk = 2 * i
        issue(blk + 1, 1)
        carry = compute(blk, 0, carry)

        @pl.when(i < nb // 2 - 1)
        def _():
            issue(blk + 2, 0)

        return compute(blk + 1, 1, carry)

    z = jnp.zeros((16,), jnp.float32)
    acc_s, acc_q = lax.fori_loop(0, nb // 2, pair_body, (z, z))
    for par in (0, 1):
        pltpu.make_async_copy(out_vs[par],
                              out_hbm.at[pl.ds(base0 * C, _QB * C)],
                              semos[par]).wait()
    stats_v[pl.ds(0, 16)] = acc_s
    stats_v[pl.ds(16, 16)] = acc_q
    pltpu.sync_copy(stats_v, stats_hbm.at[pl.ds(wid * 32, 32)])


def _sc_gather(idx, w, table):
    Q = idx.shape[0]
    C = table.shape[1]
    qw = Q // _NW
    nb = qw // _QB
    mesh = plsc.VectorSubcoreMesh(core_axis_name="c", subcore_axis_name="s")
    out, stats = pl.kernel(
        functools.partial(_sc_gather_body, C=C, qw=qw),
        out_type=[
            jax.ShapeDtypeStruct((Q * C,), jnp.float32),
            jax.ShapeDtypeStruct((_NW * 32,), jnp.float32),
        ],
        mesh=mesh,
        scratch_types=[
            pltpu.VMEM((nb, _QB * 3), jnp.int32),
            pltpu.VMEM((qw * 3 + 16,), jnp.float32),
            pltpu.VMEM((_QB * 3, C), jnp.float32),
            pltpu.VMEM((_QB * 3, C), jnp.float32),
            pltpu.VMEM((_QB * C,), jnp.float32),
            pltpu.VMEM((_QB * C,), jnp.float32),
            pltpu.VMEM((32,), jnp.float32),
            pltpu.SemaphoreType.DMA,
            pltpu.SemaphoreType.DMA,
            pltpu.SemaphoreType.DMA,
            pltpu.SemaphoreType.DMA,
        ],
        name="sc_gather_interp",
    )(idx.reshape(-1, _QB * 3), w.reshape(-1), table)
    st = stats.reshape(_NW, 32)
    return out.reshape(Q, C), jnp.sum(st[:, :16]), jnp.sum(st[:, 16:])


def _mean_scale(s1, s2, n):
    mean = s1 / n
    m2 = s2 - n * mean * mean
    std = jnp.sqrt(m2 / (n - 1))
    scale = 1.0 / (std + 1e-5)
    return jnp.stack([mean, scale]).astype(jnp.float32)


def _norm1_body(x1_ref, it_ref, st_ref, out_ref, *, C1):
    out_ref[:, :C1] = jnp.transpose(x1_ref[...])
    out_ref[:, C1:] = (it_ref[...] - st_ref[0]) * st_ref[1]


def _norm1(x1, interp, stats, tile_n):
    B, C1, N = x1.shape
    C2 = interp.shape[2]
    grid = (B, N // tile_n)
    return pl.pallas_call(
        functools.partial(_norm1_body, C1=C1),
        grid=grid,
        in_specs=[
            pl.BlockSpec((None, C1, tile_n), lambda b, n: (b, 0, n)),
            pl.BlockSpec((None, tile_n, C2), lambda b, n: (b, n, 0)),
            pl.BlockSpec(memory_space=pltpu.SMEM),
        ],
        out_specs=pl.BlockSpec((None, tile_n, C1 + C2), lambda b, n: (b, n, 0)),
        out_shape=jax.ShapeDtypeStruct((B, N, C1 + C2), jnp.float32),
    )(x1, interp, stats)


def _norm2_body(x0_ref, it_ref, st_ref, out_ref):
    c = pl.program_id(2)

    @pl.when(c == 0)
    def _copy():
        out_ref[...] = x0_ref[...]

    @pl.when(c > 0)
    def _norm():
        out_ref[...] = jnp.transpose((it_ref[...] - st_ref[0]) * st_ref[1])


def _norm2(x0, interp, stats, tile_n):
    B, C0, N = x0.shape
    C2 = interp.shape[2]
    ct = (C0 + C2) // C0
    grid = (B, N // tile_n, ct)
    return pl.pallas_call(
        _norm2_body,
        grid=grid,
        in_specs=[
            pl.BlockSpec((None, C0, tile_n), lambda b, n, c: (b, 0, n)),
            pl.BlockSpec((None, tile_n, C0),
                         lambda b, n, c: (b, n, jnp.maximum(c - 1, 0))),
            pl.BlockSpec(memory_space=pltpu.SMEM),
        ],
        out_specs=pl.BlockSpec((None, C0, tile_n), lambda b, n, c: (b, c, n)),
        out_shape=jax.ShapeDtypeStruct((B, C0 + C2, N), jnp.float32),
    )(x0, interp, stats)


def kernel(xyz_list_0, xyz_list_1, xyz_list_2, x_list_0, x_list_1, x_list_2):
    B, N1, _ = xyz_list_1.shape
    N2 = xyz_list_0.shape[1]
    S1 = xyz_list_2.shape[1]
    C2a = x_list_2.shape[1]

    idx1, w1 = _knn_topk(xyz_list_1, jnp.transpose(xyz_list_2, (0, 2, 1)), 512)
    idx2, w2 = _knn_topk(xyz_list_0, jnp.transpose(xyz_list_1, (0, 2, 1)), 512)

    table1 = jnp.transpose(x_list_2, (0, 2, 1)).reshape(B * S1, C2a)
    interp1, s1a, s2a = _sc_gather(idx1.reshape(-1, _K), w1.reshape(-1, _K),
                                   table1)
    st1 = _mean_scale(s1a, s2a, float(B * N1 * C2a))
    out1t = _norm1(x_list_1, interp1.reshape(B, N1, C2a), st1, 256)

    C2b = out1t.shape[2]
    interp2, s1b, s2b = _sc_gather(idx2.reshape(-1, _K), w2.reshape(-1, _K),
                                   out1t.reshape(B * N1, C2b))
    st2 = _mean_scale(s1b, s2b, float(B * N2 * C2b))
    return _norm2(x_list_0, interp2.reshape(B, N2, C2b), st2, 512)


# speedup vs baseline: 20.7248x; 1.2096x over previous
import functools

import jax
import jax.numpy as jnp
from jax import lax
from jax.experimental import pallas as pl
from jax.experimental.pallas import tpu as pltpu
from jax.experimental.pallas import tpu_sc as plsc

_K = 3
_BIG = 3.0e38
_NC, _NS = 2, 16
_NW = _NC * _NS
_QB = 16


def _knn_body(q_ref, p_ref, idx_ref, w_ref, *, S):
    q = q_ref[...]
    p = p_ref[...]
    qn = (q[:, 0:1] * q[:, 0:1] + q[:, 1:2] * q[:, 1:2]) + q[:, 2:3] * q[:, 2:3]
    pn = (p[0:1, :] * p[0:1, :] + p[1:2, :] * p[1:2, :]) + p[2:3, :] * p[2:3, :]
    cross = lax.dot_general(q.astype(jnp.bfloat16), p.astype(jnp.bfloat16),
                            (((1,), (0,)), ((), ())),
                            preferred_element_type=jnp.float32)
    d = -2.0 * cross + qn + pn
    iota = lax.broadcasted_iota(jnp.int32, d.shape, 1).astype(jnp.float32)
    idxs, vals = [], []
    for _ in range(_K):
        m = jnp.min(d, axis=1, keepdims=True)
        i = jnp.min(jnp.where(d == m, iota, _BIG), axis=1, keepdims=True)
        idxs.append(i)
        vals.append(m)
        d = jnp.where(iota == i, _BIG, d)
    r = [1.0 / (v + 1e-8) for v in vals]
    norm = r[0] + r[1] + r[2]
    base = pl.program_id(0) * S
    idx_ref[...] = jnp.concatenate(
        [i.astype(jnp.int32) + base for i in idxs], axis=1)
    w_ref[...] = jnp.concatenate([ri / norm for ri in r], axis=1)


def _knn_topk(xyz1, xyz2t, tile_n):
    B, N, _ = xyz1.shape
    S = xyz2t.shape[2]
    grid = (B, N // tile_n)
    return pl.pallas_call(
        functools.partial(_knn_body, S=S),
        grid=grid,
        in_specs=[
            pl.BlockSpec((None, tile_n, 3), lambda b, n: (b, n, 0)),
            pl.BlockSpec((None, 3, S), lambda b, n: (b, 0, 0)),
        ],
        out_specs=[
            pl.BlockSpec((None, tile_n, _K), lambda b, n: (b, n, 0)),
            pl.BlockSpec((None, tile_n, _K), lambda b, n: (b, n, 0)),
        ],
        out_shape=[
            jax.ShapeDtypeStruct((B, N, _K), jnp.int32),
            jax.ShapeDtypeStruct((B, N, _K), jnp.float32),
        ],
    )(xyz1, xyz2t)


def _sc_gather_body(idx_hbm, w_hbm, table_hbm, out_hbm, stats_hbm,
                    idxall_v, wall_v, rows_v0, rows_v1, out_v0, out_v1,
                    stats_v, semg0, semg1, semo0, semo1, *, C, qw):
    wid = lax.axis_index("s") * _NC + lax.axis_index("c")
    nb = qw // _QB
    base0 = wid * qw
    rows_vs = (rows_v0, rows_v1)
    out_vs = (out_v0, out_v1)
    semgs = (semg0, semg1)
    semos = (semo0, semo1)

    pltpu.sync_copy(idx_hbm.at[pl.ds(wid * nb, nb)], idxall_v)
    pltpu.sync_copy(w_hbm.at[pl.ds(base0 * 3, qw * 3)],
                    wall_v.at[pl.ds(0, qw * 3)])

    def issue(blk, par):
        pltpu.async_copy(table_hbm.at[idxall_v.at[blk]], rows_vs[par],
                         semgs[par])

    def compute(blk, par, carry):
        base = base0 + blk * _QB
        pltpu.make_async_copy(table_hbm.at[idxall_v.at[blk]], rows_vs[par],
                              semgs[par]).wait()
        rows_v = rows_vs[par]
        out_v = out_vs[par]

        @pl.when(blk >= 2)
        def _():
            pltpu.make_async_copy(out_v, out_hbm.at[pl.ds(base * C, _QB * C)],
                                  semos[par]).wait()

        def q_body(q, carry2):
            a_s, a_q = carry2
            wtri = wall_v[pl.ds(blk * (3 * _QB) + 3 * q, 16)]
            w0 = wtri[0]
            w1 = wtri[1]
            w2 = wtri[2]
            for c in range(C // 16):
                r0 = rows_v[3 * q, pl.ds(c * 16, 16)]
                r1 = rows_v[3 * q + 1, pl.ds(c * 16, 16)]
                r2 = rows_v[3 * q + 2, pl.ds(c * 16, 16)]
                y = r0 * w0 + r1 * w1 + r2 * w2
                out_v[pl.ds(q * C + c * 16, 16)] = y
                a_s = a_s + y
                a_q = a_q + y * y
            return a_s, a_q

        carry = lax.fori_loop(0, _QB, q_body, carry)
        pltpu.async_copy(out_v, out_hbm.at[pl.ds(base * C, _QB * C)],
                         semos[par])
        return carry

    issue(0, 0)

    def pair_body(i, carry):
        blk = 2 * i
        issue(blk + 1, 1)
        carry = compute(blk, 0, carry)

        @pl.when(i < nb // 2 - 1)
        def _():
            issue(blk + 2, 0)

        return compute(blk + 1, 1, carry)

    z = jnp.zeros((16,), jnp.float32)
    acc_s, acc_q = lax.fori_loop(0, nb // 2, pair_body, (z, z))
    for par in (0, 1):
        pltpu.make_async_copy(out_vs[par],
                              out_hbm.at[pl.ds(base0 * C, _QB * C)],
                              semos[par]).wait()
    stats_v[pl.ds(0, 16)] = acc_s
    stats_v[pl.ds(16, 16)] = acc_q
    pltpu.sync_copy(stats_v, stats_hbm.at[pl.ds(wid * 32, 32)])


def _sc_gather(idx, w, table):
    Q = idx.shape[0]
    C = table.shape[1]
    qw = Q // _NW
    nb = qw // _QB
    mesh = plsc.VectorSubcoreMesh(core_axis_name="c", subcore_axis_name="s")
    out, stats = pl.kernel(
        functools.partial(_sc_gather_body, C=C, qw=qw),
        out_type=[
            jax.ShapeDtypeStruct((Q * C,), jnp.float32),
            jax.ShapeDtypeStruct((_NW * 32,), jnp.float32),
        ],
        mesh=mesh,
        scratch_types=[
            pltpu.VMEM((nb, _QB * 3), jnp.int32),
            pltpu.VMEM((qw * 3 + 16,), jnp.float32),
            pltpu.VMEM((_QB * 3, C), jnp.float32),
            pltpu.VMEM((_QB * 3, C), jnp.float32),
            pltpu.VMEM((_QB * C,), jnp.float32),
            pltpu.VMEM((_QB * C,), jnp.float32),
            pltpu.VMEM((32,), jnp.float32),
            pltpu.SemaphoreType.DMA,
            pltpu.SemaphoreType.DMA,
            pltpu.SemaphoreType.DMA,
            pltpu.SemaphoreType.DMA,
        ],
        name="sc_gather_interp",
    )(idx.reshape(-1, _QB * 3), w.reshape(-1), table)
    st = stats.reshape(_NW, 32)
    return out.reshape(Q, C), jnp.sum(st[:, :16]), jnp.sum(st[:, 16:])


def _mean_scale(s1, s2, n):
    mean = s1 / n
    m2 = s2 - n * mean * mean
    std = jnp.sqrt(m2 / (n - 1))
    scale = 1.0 / (std + 1e-5)
    return jnp.stack([mean, scale]).astype(jnp.float32)


def _interp_body(idx_ref, w_ref, p2t_ref, out_ref, s1_ref, s2_ref, *, S):
    gidx = idx_ref[...]
    w = w_ref[...]
    base = pl.program_id(0) * S
    iota = lax.broadcasted_iota(jnp.int32, (gidx.shape[0], S), 1) + base
    W = jnp.where(iota == gidx[:, 0:1], w[:, 0:1], 0.0)
    W = W + jnp.where(iota == gidx[:, 1:2], w[:, 1:2], 0.0)
    W = W + jnp.where(iota == gidx[:, 2:3], w[:, 2:3], 0.0)
    out = lax.dot_general(W, p2t_ref[...], (((1,), (0,)), ((), ())),
                          preferred_element_type=jnp.float32)
    out_ref[...] = out
    s1_ref[...] = jnp.full((1, 128), jnp.sum(out), jnp.float32)
    s2_ref[...] = jnp.full((1, 128), jnp.sum(out * out), jnp.float32)


def _interp_matmul(idx, w, p2t, tile_n):
    B, N, _ = idx.shape
    S, C = p2t.shape[1], p2t.shape[2]
    nt = N // tile_n
    out, s1, s2 = pl.pallas_call(
        functools.partial(_interp_body, S=S),
        grid=(B, nt),
        in_specs=[
            pl.BlockSpec((None, tile_n, _K), lambda b, n: (b, n, 0)),
            pl.BlockSpec((None, tile_n, _K), lambda b, n: (b, n, 0)),
            pl.BlockSpec((None, S, C), lambda b, n: (b, 0, 0)),
        ],
        out_specs=[
            pl.BlockSpec((None, tile_n, C), lambda b, n: (b, n, 0)),
            pl.BlockSpec((None, None, 1, 128), lambda b, n: (b, n, 0, 0)),
            pl.BlockSpec((None, None, 1, 128), lambda b, n: (b, n, 0, 0)),
        ],
        out_shape=[
            jax.ShapeDtypeStruct((B, N, C), jnp.float32),
            jax.ShapeDtypeStruct((B, nt, 1, 128), jnp.float32),
            jax.ShapeDtypeStruct((B, nt, 1, 128), jnp.float32),
        ],
    )(idx, w, p2t)
    return out, jnp.sum(s1[..., 0]), jnp.sum(s2[..., 0])


def _norm1_body(x1_ref, it_ref, st_ref, out_ref, *, C1):
    out_ref[:, :C1] = jnp.transpose(x1_ref[...])
    out_ref[:, C1:] = (it_ref[...] - st_ref[0]) * st_ref[1]


def _norm1(x1, interp, stats, tile_n):
    B, C1, N = x1.shape
    C2 = interp.shape[2]
    grid = (B, N // tile_n)
    return pl.pallas_call(
        functools.partial(_norm1_body, C1=C1),
        grid=grid,
        in_specs=[
            pl.BlockSpec((None, C1, tile_n), lambda b, n: (b, 0, n)),
            pl.BlockSpec((None, tile_n, C2), lambda b, n: (b, n, 0)),
            pl.BlockSpec(memory_space=pltpu.SMEM),
        ],
        out_specs=pl.BlockSpec((None, tile_n, C1 + C2), lambda b, n: (b, n, 0)),
        out_shape=jax.ShapeDtypeStruct((B, N, C1 + C2), jnp.float32),
    )(x1, interp, stats)


def _norm2_body(x0_ref, isc_ref, itc_ref, st_ref, out_ref, *, nt_sc):
    n = pl.program_id(1)
    c = pl.program_id(2)

    @pl.when(c == 0)
    def _copy():
        out_ref[...] = x0_ref[...]

    @pl.when((c > 0) & (n < nt_sc))
    def _norm_sc():
        out_ref[...] = jnp.transpose((isc_ref[...] - st_ref[0]) * st_ref[1])

    @pl.when((c > 0) & (n >= nt_sc))
    def _norm_tc():
        out_ref[...] = jnp.transpose((itc_ref[...] - st_ref[0]) * st_ref[1])


def _norm2(x0, interp_sc, interp_tc, stats, tile_n):
    B, C0, N = x0.shape
    C2 = interp_sc.shape[2]
    nt_sc = interp_sc.shape[1] // tile_n
    ct = (C0 + C2) // C0
    grid = (B, N // tile_n, ct)
    return pl.pallas_call(
        functools.partial(_norm2_body, nt_sc=nt_sc),
        grid=grid,
        in_specs=[
            pl.BlockSpec((None, C0, tile_n), lambda b, n, c: (b, 0, n)),
            pl.BlockSpec((None, tile_n, C0),
                         lambda b, n, c: (b, jnp.minimum(n, nt_sc - 1),
                                          jnp.maximum(c - 1, 0))),
            pl.BlockSpec((None, tile_n, C0),
                         lambda b, n, c: (b, jnp.maximum(n - nt_sc, 0),
                                          jnp.maximum(c - 1, 0))),
            pl.BlockSpec(memory_space=pltpu.SMEM),
        ],
        out_specs=pl.BlockSpec((None, C0, tile_n), lambda b, n, c: (b, c, n)),
        out_shape=jax.ShapeDtypeStruct((B, C0 + C2, N), jnp.float32),
    )(x0, interp_sc, interp_tc, stats)


def kernel(xyz_list_0, xyz_list_1, xyz_list_2, x_list_0, x_list_1, x_list_2):
    B, N1, _ = xyz_list_1.shape
    N2 = xyz_list_0.shape[1]
    S1 = xyz_list_2.shape[1]
    C2a = x_list_2.shape[1]

    idx1, w1 = _knn_topk(xyz_list_1, jnp.transpose(xyz_list_2, (0, 2, 1)), 512)
    idx2, w2 = _knn_topk(xyz_list_0, jnp.transpose(xyz_list_1, (0, 2, 1)), 512)

    table1 = jnp.transpose(x_list_2, (0, 2, 1)).reshape(B * S1, C2a)
    interp1, s1a, s2a = _sc_gather(idx1.reshape(-1, _K), w1.reshape(-1, _K),
                                   table1)
    st1 = _mean_scale(s1a, s2a, float(B * N1 * C2a))
    out1t = _norm1(x_list_1, interp1.reshape(B, N1, C2a), st1, 256)

    C2b = out1t.shape[2]
    n_sc = 4096
    interp2sc, s1b, s2b = _sc_gather(idx2[:, :n_sc].reshape(-1, _K),
                                     w2[:, :n_sc].reshape(-1, _K),
                                     out1t.reshape(B * N1, C2b))
    interp2tc, s1c, s2c = _interp_matmul(idx2[:, n_sc:], w2[:, n_sc:],
                                         out1t, 512)
    st2 = _mean_scale(s1b + s1c, s2b + s2c, float(B * N2 * C2b))
    return _norm2(x_list_0, interp2sc.reshape(B, n_sc, C2b), interp2tc,
                  st2, 512)
